# baseline (speedup 1.0000x reference)
import functools

import jax
import jax.numpy as jnp
from jax import lax
from jax.experimental import pallas as pl
from jax.experimental.pallas import tpu as pltpu
from jax.experimental.pallas import tpu_sc as plsc

NUM_CORES = 2
NUM_SUBCORES = 16
LANES = 16
NW = NUM_CORES * NUM_SUBCORES

BATCH = 16384
EMBED = 64
BPW = BATCH // NW
CHUNK = 128
NCHUNK = BPW // CHUNK


def _bpr_body(user_hbm, pos_hbm, neg_hbm, ut_hbm, it_hbm,
              outp_hbm, outn_hbm,
              idx_u, idx_p, idx_n, rows_u, rows_p, rows_n,
              outp_v, outn_v, sem):
    wid = lax.axis_index("s") * NUM_CORES + lax.axis_index("c")
    base = wid * BPW

    pltpu.sync_copy(user_hbm.at[wid], idx_u)
    pltpu.sync_copy(pos_hbm.at[wid], idx_p)
    pltpu.sync_copy(neg_hbm.at[wid], idx_n)

    copies = []
    for j in range(NCHUNK):
        dst = pl.ds(j * CHUNK, CHUNK)
        copies.append(pltpu.async_copy(ut_hbm.at[idx_u.at[j]], rows_u.at[dst], sem))
        copies.append(pltpu.async_copy(it_hbm.at[idx_p.at[j]], rows_p.at[dst], sem))
        copies.append(pltpu.async_copy(it_hbm.at[idx_n.at[j]], rows_n.at[dst], sem))
    for c in copies:
        c.wait()

    lanes = lax.iota(jnp.int32, LANES)
    zeros = jnp.zeros((LANES,), jnp.float32)
    for g in range(BPW // LANES):
        rid = g * LANES + lanes

        def dbody(d, carry, rid=rid):
            ap, an = carry
            dcol = jnp.full((LANES,), d, jnp.int32)
            u = plsc.load_gather(rows_u, [rid, dcol])
            p = plsc.load_gather(rows_p, [rid, dcol])
            n = plsc.load_gather(rows_n, [rid, dcol])
            return ap + u * p, an + u * n

        ap, an = lax.fori_loop(0, EMBED, dbody, (zeros, zeros), unroll=8)
        outp_v[pl.ds(g * LANES, LANES)] = ap
        outn_v[pl.ds(g * LANES, LANES)] = an

    pltpu.sync_copy(outp_v, outp_hbm.at[pl.ds(base, BPW)])
    pltpu.sync_copy(outn_v, outn_hbm.at[pl.ds(base, BPW)])


@jax.jit
def _bpr_sc(user3, pos3, neg3, user_table, item_table):
    mesh = plsc.VectorSubcoreMesh(core_axis_name="c", subcore_axis_name="s",
                                  num_cores=NUM_CORES, num_subcores=NUM_SUBCORES)
    score = jax.ShapeDtypeStruct((BATCH,), jnp.float32)
    return pl.kernel(
        _bpr_body,
        out_type=(score, score),
        mesh=mesh,
        compiler_params=pltpu.CompilerParams(needs_layout_passes=False,
                                             use_tc_tiling_on_sc=False),
        scratch_types=[
            pltpu.VMEM((NCHUNK, CHUNK), jnp.int32),
            pltpu.VMEM((NCHUNK, CHUNK), jnp.int32),
            pltpu.VMEM((NCHUNK, CHUNK), jnp.int32),
            pltpu.VMEM((BPW, EMBED), jnp.float32),
            pltpu.VMEM((BPW, EMBED), jnp.float32),
            pltpu.VMEM((BPW, EMBED), jnp.float32),
            pltpu.VMEM((BPW,), jnp.float32),
            pltpu.VMEM((BPW,), jnp.float32),
            pltpu.SemaphoreType.DMA,
        ],
    )(user3, pos3, neg3, user_table, item_table)


def kernel(user, pos_item, neg_item, user_table, item_table):
    user3 = user.astype(jnp.int32).reshape(NW, NCHUNK, CHUNK)
    pos3 = pos_item.astype(jnp.int32).reshape(NW, NCHUNK, CHUNK)
    neg3 = neg_item.astype(jnp.int32).reshape(NW, NCHUNK, CHUNK)
    return _bpr_sc(user3, pos3, neg3, user_table, item_table)


# speedup vs baseline: 1.5671x; 1.5671x over previous
import jax
import jax.numpy as jnp
from jax import lax
from jax.experimental import pallas as pl
from jax.experimental.pallas import tpu as pltpu
from jax.experimental.pallas import tpu_sc as plsc

NUM_CORES = 2
NUM_SUBCORES = 16
LANES = 16
NW = NUM_CORES * NUM_SUBCORES

BATCH = 16384
EMBED = 64
BPW = BATCH // NW
CH = 128
NCH = BPW // CH


def _bpr_body(user_hbm, pos_hbm, neg_hbm, ut_hbm, it_hbm, dummy_hbm,
              outp_hbm, outn_hbm,
              sid_u, sid_p, sid_n,
              rows_u0, rows_p0, rows_n0, rows_u1, rows_p1, rows_n1,
              outp_v, outn_v, sem0, sem1):
    wid = lax.axis_index("s") * NUM_CORES + lax.axis_index("c")
    base = wid * BPW

    rows = ((rows_u0, rows_p0, rows_n0), (rows_u1, rows_p1, rows_n1))
    sems = (sem0, sem1)

    pltpu.sync_copy(user_hbm.at[wid], sid_u)
    pltpu.sync_copy(pos_hbm.at[wid], sid_p)
    pltpu.sync_copy(neg_hbm.at[wid], sid_n)

    def fire_chunk(c, buf):
        rows_u, rows_p, rows_n = rows[buf]
        sem = sems[buf]

        def fire(g, carry):
            v_u = sid_u[pl.ds(c * CH + g * LANES, LANES)]
            v_p = sid_p[pl.ds(c * CH + g * LANES, LANES)]
            v_n = sid_n[pl.ds(c * CH + g * LANES, LANES)]
            for j in range(LANES):
                dst = (pl.ds(g * LANES + j, 1), slice(None))
                pltpu.async_copy(ut_hbm.at[pl.ds(v_u[j], 1), :], rows_u.at[dst], sem)
                pltpu.async_copy(it_hbm.at[pl.ds(v_p[j], 1), :], rows_p.at[dst], sem)
                pltpu.async_copy(it_hbm.at[pl.ds(v_n[j], 1), :], rows_n.at[dst], sem)
            return carry

        lax.fori_loop(0, CH // LANES, fire, 0)

    lanes = lax.iota(jnp.int32, LANES)
    zeros = jnp.zeros((LANES,), jnp.float32)

    fire_chunk(0, 0)
    for c in range(NCH):
        buf = c % 2
        if c + 1 < NCH:
            fire_chunk(c + 1, (c + 1) % 2)
        rows_u, rows_p, rows_n = rows[buf]
        pltpu.make_async_copy(dummy_hbm, rows_u, sems[buf]).wait()
        pltpu.make_async_copy(dummy_hbm, rows_p, sems[buf]).wait()
        pltpu.make_async_copy(dummy_hbm, rows_n, sems[buf]).wait()

        for g in range(CH // LANES):
            rid = g * LANES + lanes

            def dbody(d, carry, rid=rid, rows_u=rows_u, rows_p=rows_p,
                      rows_n=rows_n):
                ap, an = carry
                dcol = jnp.full((LANES,), d, jnp.int32)
                u = plsc.load_gather(rows_u, [rid, dcol])
                p = plsc.load_gather(rows_p, [rid, dcol])
                n = plsc.load_gather(rows_n, [rid, dcol])
                return ap + u * p, an + u * n

            ap, an = lax.fori_loop(0, EMBED, dbody, (zeros, zeros), unroll=8)
            out_off = c * CH + g * LANES
            outp_v[pl.ds(out_off, LANES)] = ap
            outn_v[pl.ds(out_off, LANES)] = an

    pltpu.sync_copy(outp_v, outp_hbm.at[pl.ds(base, BPW)])
    pltpu.sync_copy(outn_v, outn_hbm.at[pl.ds(base, BPW)])


@jax.jit
def _bpr_sc(user2, pos2, neg2, user_table, item_table, dummy):
    mesh = plsc.VectorSubcoreMesh(core_axis_name="c", subcore_axis_name="s",
                                  num_cores=NUM_CORES, num_subcores=NUM_SUBCORES)
    score = jax.ShapeDtypeStruct((BATCH,), jnp.float32)
    return pl.kernel(
        _bpr_body,
        out_type=(score, score),
        mesh=mesh,
        compiler_params=pltpu.CompilerParams(needs_layout_passes=False),
        scratch_types=[
            pltpu.VMEM((BPW,), jnp.int32),
            pltpu.VMEM((BPW,), jnp.int32),
            pltpu.VMEM((BPW,), jnp.int32),
            pltpu.VMEM((CH, EMBED), jnp.float32),
            pltpu.VMEM((CH, EMBED), jnp.float32),
            pltpu.VMEM((CH, EMBED), jnp.float32),
            pltpu.VMEM((CH, EMBED), jnp.float32),
            pltpu.VMEM((CH, EMBED), jnp.float32),
            pltpu.VMEM((CH, EMBED), jnp.float32),
            pltpu.VMEM((BPW,), jnp.float32),
            pltpu.VMEM((BPW,), jnp.float32),
            pltpu.SemaphoreType.DMA,
            pltpu.SemaphoreType.DMA,
        ],
    )(user2, pos2, neg2, user_table, item_table, dummy)


def kernel(user, pos_item, neg_item, user_table, item_table):
    user2 = user.astype(jnp.int32).reshape(NW, BPW)
    pos2 = pos_item.astype(jnp.int32).reshape(NW, BPW)
    neg2 = neg_item.astype(jnp.int32).reshape(NW, BPW)
    dummy = jnp.zeros((CH, EMBED), jnp.float32)
    return _bpr_sc(user2, pos2, neg2, user_table, item_table, dummy)
